# baseline (speedup 1.0000x reference)
import functools

import jax
import jax.numpy as jnp
from jax import lax
from jax.experimental import pallas as pl
from jax.experimental.pallas import tpu as pltpu
from jax.experimental.pallas import tpu_sc as plsc

EMB_DIM = 128
N_EDGES = 320000
T_ROWS = 64
NUM_CORES = 2
NUM_SUBCORES = 16
NUM_WORKERS = NUM_CORES * NUM_SUBCORES
LANES = 16
CHUNK = 80
NSETS = 8
PREF = 4

N_SC = 153600
N_TC = N_EDGES - N_SC
BPW = N_SC // NUM_WORKERS
NCHUNKS = BPW // CHUNK
NBODY = -(-(NCHUNKS + PREF) // NSETS) * NSETS
R_TC = 512


def _table_body(w0_ref, w1_ref, w2_ref, t_ref):
    t_ref[...] = jnp.zeros((T_ROWS, EMB_DIM), jnp.float32)
    for i in range(5):
        for j in range(6):
            for k in range(2):
                r = (i * 6 + j) * 2 + k
                t_ref[pl.ds(r, 1), :] = (
                    w0_ref[pl.ds(i, 1), :]
                    + w1_ref[pl.ds(j, 1), :]
                    + w2_ref[pl.ds(k, 1), :]
                )


def _build_table(W0, W1, W2):
    return pl.pallas_call(
        _table_body,
        out_shape=jax.ShapeDtypeStruct((T_ROWS, EMB_DIM), jnp.float32),
    )(W0, W1, W2)


def _sc_lookup(a0, a1, a2, table):
    mesh = plsc.VectorSubcoreMesh(core_axis_name="c", subcore_axis_name="s")

    @functools.partial(
        pl.kernel,
        mesh=mesh,
        out_type=jax.ShapeDtypeStruct((N_SC, EMB_DIM), jnp.float32),
        scratch_types=[
            pltpu.VMEM((BPW,), jnp.int32),
            pltpu.VMEM((BPW,), jnp.int32),
            pltpu.VMEM((BPW,), jnp.int32),
            pltpu.VMEM((BPW,), jnp.int32),
            pltpu.VMEM_SHARED((T_ROWS, EMB_DIM), jnp.float32),
        ]
        + [pltpu.VMEM((CHUNK, EMB_DIM), jnp.float32) for _ in range(NSETS)]
        + [pltpu.SemaphoreType.DMA for _ in range(2 * NSETS)],
    )
    def body(a0_hbm, a1_hbm, a2_hbm, t_hbm, out_hbm, c0_v, c1_v, c2_v,
             codes_v, t_sh, *rest):
        bufs = rest[:NSETS]
        gsems = rest[NSETS:2 * NSETS]
        ssems = rest[2 * NSETS:]
        wid = lax.axis_index("s") * NUM_CORES + lax.axis_index("c")
        base = wid * BPW

        @pl.when(lax.axis_index("s") == 0)
        def _():
            pltpu.sync_copy(t_hbm, t_sh)

        cp0 = pltpu.async_copy(a0_hbm.at[pl.ds(base, BPW)], c0_v, gsems[0])
        cp1 = pltpu.async_copy(a1_hbm.at[pl.ds(base, BPW)], c1_v, gsems[1])
        cp2 = pltpu.async_copy(a2_hbm.at[pl.ds(base, BPW)], c2_v, gsems[2])
        cp0.wait()
        cp1.wait()
        cp2.wait()

        def code_body(j, _):
            o = pl.multiple_of(j * LANES, LANES)
            codes_v[pl.ds(o, LANES)] = (
                c0_v[pl.ds(o, LANES)] * 12
                + c1_v[pl.ds(o, LANES)] * 2
                + c2_v[pl.ds(o, LANES)]
            )
            return 0

        lax.fori_loop(0, BPW // LANES, code_body, 0, unroll=4)
        plsc.subcore_barrier()

        def fire_gather(i, p):
            off = pl.multiple_of(i * CHUNK, CHUNK)
            idx = codes_v.at[pl.ds(off, CHUNK)]
            pltpu.async_copy(t_sh.at[idx], bufs[p], gsems[p])

        def fire_scatter(i, p):
            off = pl.multiple_of(i * CHUNK, CHUNK)
            pltpu.async_copy(bufs[p], out_hbm.at[pl.ds(base + off, CHUNK)],
                             ssems[p])

        def drain_gather(p):
            pltpu.make_async_copy(out_hbm.at[pl.ds(0, CHUNK)], bufs[p],
                                  gsems[p]).wait()

        def drain_scatter(p):
            pltpu.make_async_copy(bufs[p], out_hbm.at[pl.ds(0, CHUNK)],
                                  ssems[p]).wait()

        for c in range(PREF):
            fire_gather(c, c)

        def super_body(s, _):
            for p in range(NSETS):
                g = s * NSETS + p
                sp = (p + PREF) % NSETS

                @pl.when((g >= PREF) & (g < NCHUNKS + PREF))
                def _():
                    drain_scatter(sp)

                @pl.when(g + PREF < NCHUNKS)
                def _():
                    fire_gather(g + PREF, sp)

                @pl.when(g < NCHUNKS)
                def _():
                    drain_gather(p)
                    fire_scatter(g, p)

            return 0

        lax.fori_loop(0, NBODY // NSETS, super_body, 0)

    return body(a0, a1, a2, table)


def _tc_lookup_body(a0_ref, a1_ref, a2_ref, w0_ref, w1_ref, w2_ref, o_ref):
    cols = lax.broadcasted_iota(jnp.int32, (R_TC, 8), 1)
    oh0 = (cols == a0_ref[0, 0, :][:, None]).astype(jnp.float32)
    oh1 = (cols == a1_ref[0, 0, :][:, None]).astype(jnp.float32)
    oh2 = (cols == a2_ref[0, 0, :][:, None]).astype(jnp.float32)
    o_ref[...] = (
        jnp.dot(oh0, w0_ref[...], preferred_element_type=jnp.float32,
                  precision=lax.Precision.HIGHEST)
        + jnp.dot(oh1, w1_ref[...], preferred_element_type=jnp.float32,
                  precision=lax.Precision.HIGHEST)
        + jnp.dot(oh2, w2_ref[...], preferred_element_type=jnp.float32,
                  precision=lax.Precision.HIGHEST)
    )


def _tc_lookup(a0, a1, a2, W0p, W1p, W2p):
    nb = N_TC // R_TC
    idx_spec = pl.BlockSpec((1, 1, R_TC), lambda i: (i, 0, 0))
    w_spec = pl.BlockSpec((8, EMB_DIM), lambda i: (0, 0))
    return pl.pallas_call(
        _tc_lookup_body,
        grid=(nb,),
        in_specs=[idx_spec, idx_spec, idx_spec, w_spec, w_spec, w_spec],
        out_specs=pl.BlockSpec((R_TC, EMB_DIM), lambda i: (i, 0)),
        out_shape=jax.ShapeDtypeStruct((N_TC, EMB_DIM), jnp.float32),
    )(a0.reshape(nb, 1, R_TC), a1.reshape(nb, 1, R_TC),
      a2.reshape(nb, 1, R_TC), W0p, W1p, W2p)


def kernel(edge_attr, W0, W1, W2):
    table = _build_table(W0, W1, W2)
    a0 = edge_attr[:, 0]
    a1 = edge_attr[:, 1]
    a2 = edge_attr[:, 2]
    sc_out = _sc_lookup(a0[:N_SC], a1[:N_SC], a2[:N_SC], table)
    pad = jnp.zeros((8, EMB_DIM), jnp.float32)
    W0p = pad.at[:5].set(W0)
    W1p = pad.at[:6].set(W1)
    W2p = pad.at[:2].set(W2)
    tc_out = _tc_lookup(a0[N_SC:], a1[N_SC:], a2[N_SC:], W0p, W1p, W2p)
    return jnp.concatenate([sc_out, tc_out], axis=0)


# speedup vs baseline: 2.8132x; 2.8132x over previous
import functools

import jax
import jax.numpy as jnp
from jax import lax
from jax.experimental import pallas as pl
from jax.experimental.pallas import tpu as pltpu
from jax.experimental.pallas import tpu_sc as plsc

EMB_DIM = 128
N_EDGES = 320000
T_ROWS = 64
NUM_CORES = 2
NUM_SUBCORES = 16
NUM_WORKERS = NUM_CORES * NUM_SUBCORES
LANES = 16
CHUNK = 80
NSETS = 8
PREF = 4

N_SC = 153600
N_TC = N_EDGES - N_SC
BPW = N_SC // NUM_WORKERS
NCHUNKS = BPW // CHUNK
NBODY = -(-(NCHUNKS + PREF) // NSETS) * NSETS
R_TC = 1280


def _table_body(w0_ref, w1_ref, w2_ref, t_ref):
    t_ref[...] = jnp.zeros((T_ROWS, EMB_DIM), jnp.float32)
    for i in range(5):
        for j in range(6):
            for k in range(2):
                r = (i * 6 + j) * 2 + k
                t_ref[pl.ds(r, 1), :] = (
                    w0_ref[pl.ds(i, 1), :]
                    + w1_ref[pl.ds(j, 1), :]
                    + w2_ref[pl.ds(k, 1), :]
                )


def _build_table(W0, W1, W2):
    return pl.pallas_call(
        _table_body,
        out_shape=jax.ShapeDtypeStruct((T_ROWS, EMB_DIM), jnp.float32),
    )(W0, W1, W2)


def _sc_lookup(a0, a1, a2, table):
    mesh = plsc.VectorSubcoreMesh(core_axis_name="c", subcore_axis_name="s")

    @functools.partial(
        pl.kernel,
        mesh=mesh,
        out_type=jax.ShapeDtypeStruct((N_EDGES, EMB_DIM), jnp.float32),
        scratch_types=[
            pltpu.VMEM((BPW,), jnp.int32),
            pltpu.VMEM((BPW,), jnp.int32),
            pltpu.VMEM((BPW,), jnp.int32),
            pltpu.VMEM((BPW,), jnp.int32),
            pltpu.VMEM_SHARED((T_ROWS, EMB_DIM), jnp.float32),
        ]
        + [pltpu.VMEM((CHUNK, EMB_DIM), jnp.float32) for _ in range(NSETS)]
        + [pltpu.SemaphoreType.DMA for _ in range(2 * NSETS)],
    )
    def body(a0_hbm, a1_hbm, a2_hbm, t_hbm, out_hbm, c0_v, c1_v, c2_v,
             codes_v, t_sh, *rest):
        bufs = rest[:NSETS]
        gsems = rest[NSETS:2 * NSETS]
        ssems = rest[2 * NSETS:]
        wid = lax.axis_index("s") * NUM_CORES + lax.axis_index("c")
        base = wid * BPW

        @pl.when(lax.axis_index("s") == 0)
        def _():
            pltpu.sync_copy(t_hbm, t_sh)

        cp0 = pltpu.async_copy(a0_hbm.at[pl.ds(base, BPW)], c0_v, gsems[0])
        cp1 = pltpu.async_copy(a1_hbm.at[pl.ds(base, BPW)], c1_v, gsems[1])
        cp2 = pltpu.async_copy(a2_hbm.at[pl.ds(base, BPW)], c2_v, gsems[2])
        cp0.wait()
        cp1.wait()
        cp2.wait()

        def code_body(j, _):
            o = pl.multiple_of(j * LANES, LANES)
            codes_v[pl.ds(o, LANES)] = (
                c0_v[pl.ds(o, LANES)] * 12
                + c1_v[pl.ds(o, LANES)] * 2
                + c2_v[pl.ds(o, LANES)]
            )
            return 0

        lax.fori_loop(0, BPW // LANES, code_body, 0, unroll=4)
        plsc.subcore_barrier()

        def fire_gather(i, p):
            off = pl.multiple_of(i * CHUNK, CHUNK)
            idx = codes_v.at[pl.ds(off, CHUNK)]
            pltpu.async_copy(t_sh.at[idx], bufs[p], gsems[p])

        def fire_scatter(i, p):
            off = pl.multiple_of(i * CHUNK, CHUNK)
            pltpu.async_copy(bufs[p], out_hbm.at[pl.ds(base + off, CHUNK)],
                             ssems[p])

        def drain_gather(p):
            pltpu.make_async_copy(out_hbm.at[pl.ds(0, CHUNK)], bufs[p],
                                  gsems[p]).wait()

        def drain_scatter(p):
            pltpu.make_async_copy(bufs[p], out_hbm.at[pl.ds(0, CHUNK)],
                                  ssems[p]).wait()

        for c in range(PREF):
            fire_gather(c, c)

        def super_body(s, _):
            for p in range(NSETS):
                g = s * NSETS + p
                sp = (p + PREF) % NSETS

                @pl.when((g >= PREF) & (g < NCHUNKS + PREF))
                def _():
                    drain_scatter(sp)

                @pl.when(g + PREF < NCHUNKS)
                def _():
                    fire_gather(g + PREF, sp)

                @pl.when(g < NCHUNKS)
                def _():
                    drain_gather(p)
                    fire_scatter(g, p)

            return 0

        lax.fori_loop(0, NBODY // NSETS, super_body, 0)

    return body(a0, a1, a2, table)


def _tc_fill_body(big_ref, a0_ref, a1_ref, a2_ref, t_ref, o_ref):
    del big_ref
    codes = (a0_ref[0, 0, :] * 12 + a1_ref[0, 0, :] * 2 + a2_ref[0, 0, :])
    oh = (lax.broadcasted_iota(jnp.int32, (R_TC, T_ROWS), 1)
          == codes[:, None]).astype(jnp.bfloat16)
    t = t_ref[...]
    t_hi = t.astype(jnp.bfloat16)
    t_lo = (t - t_hi.astype(jnp.float32)).astype(jnp.bfloat16)
    o_ref[...] = (
        jnp.dot(oh, t_hi, preferred_element_type=jnp.float32)
        + jnp.dot(oh, t_lo, preferred_element_type=jnp.float32)
    )


def _tc_fill(big, a0, a1, a2, table):
    nb = N_TC // R_TC
    idx_spec = pl.BlockSpec((1, 1, R_TC), lambda i: (i, 0, 0))
    return pl.pallas_call(
        _tc_fill_body,
        grid=(nb,),
        in_specs=[
            pl.BlockSpec(memory_space=pl.ANY),
            idx_spec, idx_spec, idx_spec,
            pl.BlockSpec((T_ROWS, EMB_DIM), lambda i: (0, 0)),
        ],
        out_specs=pl.BlockSpec((R_TC, EMB_DIM),
                               lambda i: (N_SC // R_TC + i, 0)),
        out_shape=jax.ShapeDtypeStruct((N_EDGES, EMB_DIM), jnp.float32),
        input_output_aliases={0: 0},
    )(big, a0.reshape(nb, 1, R_TC), a1.reshape(nb, 1, R_TC),
      a2.reshape(nb, 1, R_TC), table)


def kernel(edge_attr, W0, W1, W2):
    table = _build_table(W0, W1, W2)
    a0 = edge_attr[:, 0]
    a1 = edge_attr[:, 1]
    a2 = edge_attr[:, 2]
    big = _sc_lookup(a0[:N_SC], a1[:N_SC], a2[:N_SC], table)
    return _tc_fill(big, a0[N_SC:], a1[N_SC:], a2[N_SC:], table)


# speedup vs baseline: 4.1500x; 1.4752x over previous
import functools

import jax
import jax.numpy as jnp
from jax import lax
from jax.experimental import pallas as pl
from jax.experimental.pallas import tpu as pltpu
from jax.experimental.pallas import tpu_sc as plsc

EMB_DIM = 128
N_EDGES = 320000
T_ROWS = 64
NUM_CORES = 2
NUM_SUBCORES = 16
NUM_WORKERS = NUM_CORES * NUM_SUBCORES
LANES = 16
CHUNK = 80
NSETS = 8
PREF = 4

BPW = N_EDGES // NUM_WORKERS
NCHUNKS = BPW // CHUNK
NBODY = -(-(NCHUNKS + PREF) // NSETS) * NSETS


def _table_body(w0_ref, w1_ref, w2_ref, t_ref):
    t_ref[...] = jnp.zeros((T_ROWS, EMB_DIM), jnp.float32)
    for i in range(5):
        for j in range(6):
            for k in range(2):
                r = (i * 6 + j) * 2 + k
                t_ref[pl.ds(r, 1), :] = (
                    w0_ref[pl.ds(i, 1), :]
                    + w1_ref[pl.ds(j, 1), :]
                    + w2_ref[pl.ds(k, 1), :]
                )


def _build_table(W0, W1, W2):
    return pl.pallas_call(
        _table_body,
        out_shape=jax.ShapeDtypeStruct((T_ROWS, EMB_DIM), jnp.float32),
    )(W0, W1, W2)


def _sc_lookup(a0, a1, a2, table):
    mesh = plsc.VectorSubcoreMesh(core_axis_name="c", subcore_axis_name="s")

    @functools.partial(
        pl.kernel,
        mesh=mesh,
        out_type=jax.ShapeDtypeStruct((N_EDGES, EMB_DIM), jnp.float32),
        scratch_types=[
            pltpu.VMEM((BPW,), jnp.int32),
            pltpu.VMEM((BPW,), jnp.int32),
            pltpu.VMEM((BPW,), jnp.int32),
            pltpu.VMEM((BPW,), jnp.int32),
            pltpu.VMEM_SHARED((T_ROWS, EMB_DIM), jnp.float32),
        ]
        + [pltpu.VMEM((CHUNK, EMB_DIM), jnp.float32) for _ in range(NSETS)]
        + [pltpu.SemaphoreType.DMA for _ in range(2 * NSETS)],
    )
    def body(a0_hbm, a1_hbm, a2_hbm, t_hbm, out_hbm, c0_v, c1_v, c2_v,
             codes_v, t_sh, *rest):
        bufs = rest[:NSETS]
        gsems = rest[NSETS:2 * NSETS]
        ssems = rest[2 * NSETS:]
        wid = lax.axis_index("s") * NUM_CORES + lax.axis_index("c")
        base = wid * BPW

        @pl.when(lax.axis_index("s") == 0)
        def _():
            pltpu.sync_copy(t_hbm, t_sh)

        cp0 = pltpu.async_copy(a0_hbm.at[pl.ds(base, BPW)], c0_v, gsems[0])
        cp1 = pltpu.async_copy(a1_hbm.at[pl.ds(base, BPW)], c1_v, gsems[1])
        cp2 = pltpu.async_copy(a2_hbm.at[pl.ds(base, BPW)], c2_v, gsems[2])
        cp0.wait()
        cp1.wait()
        cp2.wait()

        def code_body(j, _):
            o = pl.multiple_of(j * LANES, LANES)
            codes_v[pl.ds(o, LANES)] = (
                c0_v[pl.ds(o, LANES)] * 12
                + c1_v[pl.ds(o, LANES)] * 2
                + c2_v[pl.ds(o, LANES)]
            )
            return 0

        lax.fori_loop(0, BPW // LANES, code_body, 0, unroll=4)
        plsc.subcore_barrier()

        def fire_gather(i, p):
            off = pl.multiple_of(i * CHUNK, CHUNK)
            idx = codes_v.at[pl.ds(off, CHUNK)]
            pltpu.async_copy(t_sh.at[idx], bufs[p], gsems[p])

        def fire_scatter(i, p):
            off = pl.multiple_of(i * CHUNK, CHUNK)
            pltpu.async_copy(bufs[p], out_hbm.at[pl.ds(base + off, CHUNK)],
                             ssems[p])

        def drain_gather(p):
            pltpu.make_async_copy(out_hbm.at[pl.ds(0, CHUNK)], bufs[p],
                                  gsems[p]).wait()

        def drain_scatter(p):
            pltpu.make_async_copy(bufs[p], out_hbm.at[pl.ds(0, CHUNK)],
                                  ssems[p]).wait()

        for c in range(PREF):
            fire_gather(c, c)

        def super_body(s, _):
            for p in range(NSETS):
                g = s * NSETS + p
                sp = (p + PREF) % NSETS

                @pl.when((g >= PREF) & (g < NCHUNKS + PREF))
                def _():
                    drain_scatter(sp)

                @pl.when(g + PREF < NCHUNKS)
                def _():
                    fire_gather(g + PREF, sp)

                @pl.when(g < NCHUNKS)
                def _():
                    drain_gather(p)
                    fire_scatter(g, p)

            return 0

        lax.fori_loop(0, NBODY // NSETS, super_body, 0)

    return body(a0, a1, a2, table)


def kernel(edge_attr, W0, W1, W2):
    table = _build_table(W0, W1, W2)
    a0 = edge_attr[:, 0]
    a1 = edge_attr[:, 1]
    a2 = edge_attr[:, 2]
    return _sc_lookup(a0, a1, a2, table)


# speedup vs baseline: 4.2893x; 1.0336x over previous
import functools

import jax
import jax.numpy as jnp
from jax import lax
from jax.experimental import pallas as pl
from jax.experimental.pallas import tpu as pltpu
from jax.experimental.pallas import tpu_sc as plsc

EMB_DIM = 128
N_EDGES = 320000
T_ROWS = 64
NUM_CORES = 2
NUM_SUBCORES = 16
NUM_WORKERS = NUM_CORES * NUM_SUBCORES
LANES = 16
CHUNK = 80
NSETS = 8
PREF = 4

BPW = N_EDGES // NUM_WORKERS
NCHUNKS = BPW // CHUNK
NBODY = -(-(NCHUNKS + PREF) // NSETS) * NSETS


def _table_body(w0_ref, w1_ref, w2_ref, t_ref):
    t_ref[...] = jnp.zeros((T_ROWS, EMB_DIM), jnp.float32)
    for i in range(5):
        for j in range(6):
            for k in range(2):
                r = (i * 6 + j) * 2 + k
                t_ref[pl.ds(r, 1), :] = (
                    w0_ref[pl.ds(i, 1), :]
                    + w1_ref[pl.ds(j, 1), :]
                    + w2_ref[pl.ds(k, 1), :]
                )


def _build_table(W0, W1, W2):
    return pl.pallas_call(
        _table_body,
        out_shape=jax.ShapeDtypeStruct((T_ROWS, EMB_DIM), jnp.float32),
    )(W0, W1, W2)


def _sc_lookup(a0, a1, a2, table):
    mesh = plsc.VectorSubcoreMesh(core_axis_name="c", subcore_axis_name="s")

    @functools.partial(
        pl.kernel,
        mesh=mesh,
        out_type=jax.ShapeDtypeStruct((N_EDGES, EMB_DIM), jnp.float32),
        scratch_types=[
            pltpu.VMEM((BPW,), jnp.int32),
            pltpu.VMEM((BPW,), jnp.int32),
            pltpu.VMEM((BPW,), jnp.int32),
            pltpu.VMEM((BPW,), jnp.int32),
            pltpu.VMEM_SHARED((T_ROWS, EMB_DIM), jnp.float32),
        ]
        + [pltpu.VMEM((CHUNK, EMB_DIM), jnp.float32) for _ in range(NSETS)]
        + [pltpu.SemaphoreType.DMA for _ in range(2 * NSETS)],
    )
    def body(a0_hbm, a1_hbm, a2_hbm, t_hbm, out_hbm, c0_v, c1_v, c2_v,
             codes_v, t_sh, *rest):
        bufs = rest[:NSETS]
        gsems = rest[NSETS:2 * NSETS]
        ssems = rest[2 * NSETS:]
        wid = lax.axis_index("s") * NUM_CORES + lax.axis_index("c")
        base = wid * BPW

        @pl.when(lax.axis_index("s") == 0)
        def _():
            pltpu.sync_copy(t_hbm, t_sh)

        cp0 = pltpu.async_copy(a0_hbm.at[pl.ds(base, BPW)], c0_v, gsems[0])
        cp1 = pltpu.async_copy(a1_hbm.at[pl.ds(base, BPW)], c1_v, gsems[1])
        cp2 = pltpu.async_copy(a2_hbm.at[pl.ds(base, BPW)], c2_v, gsems[2])
        cp0.wait()
        cp1.wait()
        cp2.wait()

        def compute_codes(i):
            for u in range(CHUNK // LANES):
                o = pl.multiple_of(i * CHUNK + u * LANES, LANES)
                codes_v[pl.ds(o, LANES)] = (
                    c0_v[pl.ds(o, LANES)] * 12
                    + c1_v[pl.ds(o, LANES)] * 2
                    + c2_v[pl.ds(o, LANES)]
                )

        plsc.subcore_barrier()

        def fire_gather(i, p):
            off = pl.multiple_of(i * CHUNK, CHUNK)
            idx = codes_v.at[pl.ds(off, CHUNK)]
            pltpu.async_copy(t_sh.at[idx], bufs[p], gsems[p])

        def fire_scatter(i, p):
            off = pl.multiple_of(i * CHUNK, CHUNK)
            pltpu.async_copy(bufs[p], out_hbm.at[pl.ds(base + off, CHUNK)],
                             ssems[p])

        def drain_gather(p):
            pltpu.make_async_copy(out_hbm.at[pl.ds(0, CHUNK)], bufs[p],
                                  gsems[p]).wait()

        def drain_scatter(p):
            pltpu.make_async_copy(bufs[p], out_hbm.at[pl.ds(0, CHUNK)],
                                  ssems[p]).wait()

        for c in range(PREF):
            compute_codes(c)
            fire_gather(c, c)

        def super_body(s, _):
            for p in range(NSETS):
                g = s * NSETS + p
                sp = (p + PREF) % NSETS

                @pl.when((g >= PREF) & (g < NCHUNKS + PREF))
                def _():
                    drain_scatter(sp)

                @pl.when(g + PREF < NCHUNKS)
                def _():
                    compute_codes(g + PREF)
                    fire_gather(g + PREF, sp)

                @pl.when(g < NCHUNKS)
                def _():
                    drain_gather(p)
                    fire_scatter(g, p)

            return 0

        lax.fori_loop(0, NBODY // NSETS, super_body, 0)

    return body(a0, a1, a2, table)


def kernel(edge_attr, W0, W1, W2):
    table = _build_table(W0, W1, W2)
    a0 = edge_attr[:, 0]
    a1 = edge_attr[:, 1]
    a2 = edge_attr[:, 2]
    return _sc_lookup(a0, a1, a2, table)
